# baseline (speedup 1.0000x reference)
import jax
import jax.numpy as jnp
from jax import lax
from jax.experimental import pallas as pl
from jax.experimental.pallas import tpu as pltpu
from jax.experimental.pallas import tpu_sc as plsc

VOCAB_ = 100001
D_ = 300
B_ = 4096
L_ = 50
NC_ = 2
NS_ = 16
NW_ = NC_ * NS_
BPW_ = B_ // NW_
LANES_ = 16
SEG_ = 128
VP_ = (VOCAB_ + 7) // 8 * 8
NSEG_ = 3 * VP_
DP_ = 304
LP_ = 128
OBLK_ = DP_ * LP_
TAIL_START_ = D_ - LANES_
TAIL_SKIP_ = 288 - TAIL_START_


def _sc_body(table_hbm, tokens_hbm, out_hbm, idx_v, idx_a, idx_b, idx_c,
             rows_v, tbuf, sem):
    c = lax.axis_index("c")
    s = lax.axis_index("s")
    wid = s * NC_ + c
    base_b = wid * BPW_

    pltpu.sync_copy(tokens_hbm.at[pl.ds(base_b, BPW_)], idx_v)

    iota = lax.iota(jnp.int32, LANES_)
    iota_x_lp = iota * LP_
    tail_mask = iota >= TAIL_SKIP_

    def batch_body(i, _):
        for k in (0, LANES_, 2 * LANES_, L_ - LANES_):
            t = idx_v[i, pl.ds(k, LANES_)]
            s3 = t * 3
            sl = pl.ds(k, LANES_)
            idx_a[sl] = s3
            idx_b[sl] = s3 + 1
            idx_c[sl] = s3 + 2

        cp_a = pltpu.async_copy(table_hbm.at[idx_a], rows_v.at[pl.ds(0, L_)], sem)
        cp_b = pltpu.async_copy(table_hbm.at[idx_b], rows_v.at[pl.ds(L_, L_)], sem)
        cp_c = pltpu.async_copy(table_hbm.at[idx_c], rows_v.at[pl.ds(2 * L_, L_)], sem)
        cp_a.wait()
        cp_b.wait()
        cp_c.wait()

        def l_body(l, _):
            for j in range(18):
                d0 = j * LANES_
                blk, off = divmod(d0, SEG_)
                v = rows_v[blk * L_ + l, pl.ds(off, LANES_)]
                oidx = iota_x_lp + (d0 * LP_ + l)
                plsc.store_scatter(tbuf, [oidx], v)
            v = rows_v[2 * L_ + l, pl.ds(TAIL_START_ - 2 * SEG_, LANES_)]
            oidx = iota_x_lp + (TAIL_START_ * LP_ + l)
            plsc.store_scatter(tbuf, [oidx], v, mask=tail_mask)
            return 0

        lax.fori_loop(0, L_, l_body, 0)

        pltpu.sync_copy(tbuf, out_hbm.at[pl.ds((base_b + i) * OBLK_, OBLK_)])
        return 0

    lax.fori_loop(0, BPW_, batch_body, 0)


def kernel(tokens, word_embd_weight):
    tokens = tokens.reshape(B_, L_).astype(jnp.int32)
    tab = jnp.pad(word_embd_weight, ((0, VP_ - VOCAB_), (0, 384 - D_)))
    tab = tab.reshape(NSEG_, SEG_)
    mesh = plsc.VectorSubcoreMesh(core_axis_name="c", subcore_axis_name="s")
    out = pl.kernel(
        _sc_body,
        out_type=jax.ShapeDtypeStruct((B_ * OBLK_,), jnp.float32),
        mesh=mesh,
        scratch_types=[
            pltpu.VMEM((BPW_, L_), jnp.int32),
            pltpu.VMEM((L_,), jnp.int32),
            pltpu.VMEM((L_,), jnp.int32),
            pltpu.VMEM((L_,), jnp.int32),
            pltpu.VMEM((3 * L_, SEG_), jnp.float32),
            pltpu.VMEM((OBLK_,), jnp.float32),
            pltpu.SemaphoreType.DMA,
        ],
        compiler_params=pltpu.CompilerParams(
            needs_layout_passes=False, use_tc_tiling_on_sc=False
        ),
        name="embed_gather_transpose",
    )(tab, tokens)
    out = out.reshape(B_, DP_, LP_)
    return lax.slice(out, (0, 0, 0), (B_, D_, L_)).reshape(B_, D_, L_, 1)


# speedup vs baseline: 1.3878x; 1.3878x over previous
import jax
import jax.numpy as jnp
from jax import lax
from jax.experimental import pallas as pl
from jax.experimental.pallas import tpu as pltpu
from jax.experimental.pallas import tpu_sc as plsc

VOCAB_ = 100001
D_ = 300
B_ = 4096
L_ = 50
NC_ = 2
NS_ = 16
NW_ = NC_ * NS_
BPW_ = B_ // NW_
LANES_ = 16
SEG_ = 128
VP_ = (VOCAB_ + 7) // 8 * 8
NSEG_ = 3 * VP_
OSEG_ = D_ * L_ * NW_
TAIL_OFF_ = 284 - 2 * SEG_
TAIL_SKIP_ = 4


def _sc_body(table_hbm, tokens_hbm, out_hbm, idx_v, idx_a, idx_b, idx_c,
             rows_v, obuf, idx_o1, idx_o2, idx_o3, sem_g, sem_s):
    c = lax.axis_index("c")
    s = lax.axis_index("s")
    wid = s * NC_ + c

    pltpu.sync_copy(tokens_hbm.at[pl.ds(wid * BPW_, BPW_)], idx_v)

    iota = lax.iota(jnp.int32, LANES_)
    iota_x_o = iota * (L_ * NW_)
    tail_mask = iota >= TAIL_SKIP_
    drows = [iota + 16 * cc for cc in range(18)] + [iota + 284]

    def l_body(l, _):
        for j in range(BPW_ // LANES_):
            t = plsc.load_gather(
                idx_v, [iota + j * LANES_, jnp.broadcast_to(l, (LANES_,))])
            s3 = t * 3
            sl = pl.ds(j * LANES_, LANES_)
            idx_a[sl] = s3
            idx_b[sl] = s3 + 1
            idx_c[sl] = s3 + 2

        cp_a = pltpu.async_copy(table_hbm.at[idx_a], rows_v.at[pl.ds(0, BPW_)], sem_g)
        cp_b = pltpu.async_copy(table_hbm.at[idx_b], rows_v.at[pl.ds(BPW_, BPW_)], sem_g)
        cp_c = pltpu.async_copy(table_hbm.at[idx_c], rows_v.at[pl.ds(2 * BPW_, BPW_)], sem_g)

        obase = l * NW_ + wid
        for cc in range(8):
            idx_o1[pl.ds(cc * LANES_, LANES_)] = iota_x_o + (cc * LANES_ * L_ * NW_ + obase)
        for cc in range(8, 16):
            idx_o2[pl.ds((cc - 8) * LANES_, LANES_)] = iota_x_o + (cc * LANES_ * L_ * NW_ + obase)
        for cc, off in ((16, 0), (17, 16)):
            idx_o3[pl.ds(off, LANES_)] = iota_x_o + (cc * LANES_ * L_ * NW_ + obase)
        idx_o3[pl.ds(TAIL_OFF_, LANES_)] = iota_x_o + (284 * L_ * NW_ + obase)

        cp_a.wait()
        cp_b.wait()
        cp_c.wait()

        def t_body(jb, _):
            col = jnp.broadcast_to(jb, (LANES_,))
            for cc in range(18):
                blk, off = divmod(cc * LANES_, SEG_)
                v = rows_v[blk * BPW_ + jb, pl.ds(off, LANES_)]
                plsc.store_scatter(obuf, [drows[cc], col], v)
            v = rows_v[2 * BPW_ + jb, pl.ds(TAIL_OFF_, LANES_)]
            plsc.store_scatter(obuf, [drows[18], col], v, mask=tail_mask)
            return 0

        lax.fori_loop(0, BPW_, t_body, 0)

        cp1 = pltpu.async_copy(obuf.at[pl.ds(0, SEG_)], out_hbm.at[idx_o1], sem_s)
        cp2 = pltpu.async_copy(obuf.at[pl.ds(SEG_, SEG_)], out_hbm.at[idx_o2], sem_s)
        cp3 = pltpu.async_copy(obuf.at[pl.ds(2 * SEG_, D_ - 2 * SEG_)], out_hbm.at[idx_o3], sem_s)
        cp1.wait()
        cp2.wait()
        cp3.wait()
        return 0

    lax.fori_loop(0, L_, l_body, 0)


def kernel(tokens, word_embd_weight):
    tokens = tokens.reshape(B_, L_).astype(jnp.int32)
    tab = jnp.pad(word_embd_weight, ((0, VP_ - VOCAB_), (0, 384 - D_)))
    tab = tab.reshape(NSEG_, SEG_)
    mesh = plsc.VectorSubcoreMesh(core_axis_name="c", subcore_axis_name="s")
    out = pl.kernel(
        _sc_body,
        out_type=jax.ShapeDtypeStruct((OSEG_, SEG_), jnp.float32),
        mesh=mesh,
        scratch_types=[
            pltpu.VMEM((BPW_, L_), jnp.int32),
            pltpu.VMEM((BPW_,), jnp.int32),
            pltpu.VMEM((BPW_,), jnp.int32),
            pltpu.VMEM((BPW_,), jnp.int32),
            pltpu.VMEM((3 * BPW_, SEG_), jnp.float32),
            pltpu.VMEM((D_, SEG_), jnp.float32),
            pltpu.VMEM((SEG_,), jnp.int32),
            pltpu.VMEM((SEG_,), jnp.int32),
            pltpu.VMEM((D_ - 2 * SEG_,), jnp.int32),
            pltpu.SemaphoreType.DMA,
            pltpu.SemaphoreType.DMA,
        ],
        compiler_params=pltpu.CompilerParams(
            needs_layout_passes=False, use_tc_tiling_on_sc=False
        ),
        name="embed_gather_transpose",
    )(tab, tokens)
    out = out.reshape(D_, L_, NW_, BPW_)
    out = out.transpose(2, 3, 0, 1).reshape(B_, D_, L_, 1)
    return out


# speedup vs baseline: 1.4159x; 1.0202x over previous
import jax
import jax.numpy as jnp
from jax import lax
from jax.experimental import pallas as pl
from jax.experimental.pallas import tpu as pltpu
from jax.experimental.pallas import tpu_sc as plsc

VOCAB_ = 100001
D_ = 300
B_ = 4096
L_ = 50
NC_ = 2
NS_ = 16
NW_ = NC_ * NS_
BPW_ = B_ // NW_
LANES_ = 16
SEG_ = 128
VP_ = (VOCAB_ + 7) // 8 * 8
DPAD_ = 304
OSEG_ = D_ * L_ * NW_
TAIL_D_ = 284
TAIL_SKIP_ = 4


def _sc_body(table_hbm, tokens_hbm, out_hbm, idx_v, idx_g, rows_v,
             obuf_a, obuf_b, oi_a1, oi_a2, oi_a3, oi_b1, oi_b2, oi_b3,
             sem_g, sem_a, sem_b):
    c = lax.axis_index("c")
    s = lax.axis_index("s")
    wid = s * NC_ + c

    pltpu.sync_copy(tokens_hbm.at[pl.ds(wid * BPW_, BPW_)], idx_v)

    iota = lax.iota(jnp.int32, LANES_)
    iota_x_o = iota * (L_ * NW_)
    tail_mask = iota >= TAIL_SKIP_

    def scatter_cps(obuf, oi1, oi2, oi3, sem):
        return (
            pltpu.make_async_copy(obuf.at[pl.ds(0, SEG_)], out_hbm.at[oi1], sem),
            pltpu.make_async_copy(obuf.at[pl.ds(SEG_, SEG_)], out_hbm.at[oi2], sem),
            pltpu.make_async_copy(obuf.at[pl.ds(2 * SEG_, D_ - 2 * SEG_)], out_hbm.at[oi3], sem),
        )

    def step(l, drain, obuf, oi1, oi2, oi3, sem):
        for j in range(BPW_ // LANES_):
            t = plsc.load_gather(
                idx_v, [iota + j * LANES_, jnp.broadcast_to(l, (LANES_,))])
            idx_g[pl.ds(j * LANES_, LANES_)] = t
        cp_g = pltpu.async_copy(table_hbm.at[idx_g], rows_v, sem_g)

        @pl.when(drain)
        def _():
            for cp in scatter_cps(obuf, oi1, oi2, oi3, sem):
                cp.wait()

        obase = l * NW_ + wid
        for cc in range(8):
            oi1[pl.ds(cc * LANES_, LANES_)] = iota_x_o + (cc * LANES_ * L_ * NW_ + obase)
        for cc in range(8, 16):
            oi2[pl.ds((cc - 8) * LANES_, LANES_)] = iota_x_o + (cc * LANES_ * L_ * NW_ + obase)
        for cc, off in ((16, 0), (17, 16)):
            oi3[pl.ds(off, LANES_)] = iota_x_o + (cc * LANES_ * L_ * NW_ + obase)
        oi3[pl.ds(TAIL_D_ - 2 * SEG_, LANES_)] = iota_x_o + (TAIL_D_ * L_ * NW_ + obase)

        cp_g.wait()

        def t_body(jb, _):
            col = jnp.broadcast_to(jb, (LANES_,))
            for cc in range(18):
                v = rows_v[jb, pl.ds(cc * LANES_, LANES_)]
                plsc.store_scatter(obuf, [iota + cc * LANES_, col], v)
            v = rows_v[jb, pl.ds(TAIL_D_, LANES_)]
            plsc.store_scatter(obuf, [iota + TAIL_D_, col], v, mask=tail_mask)
            return 0

        lax.fori_loop(0, BPW_, t_body, 0)

        for cp in scatter_cps(obuf, oi1, oi2, oi3, sem):
            cp.start()

    def pair_body(i, _):
        step(2 * i, i > 0, obuf_a, oi_a1, oi_a2, oi_a3, sem_a)
        step(2 * i + 1, i > 0, obuf_b, oi_b1, oi_b2, oi_b3, sem_b)
        return 0

    lax.fori_loop(0, L_ // 2, pair_body, 0)
    for cp in scatter_cps(obuf_a, oi_a1, oi_a2, oi_a3, sem_a):
        cp.wait()
    for cp in scatter_cps(obuf_b, oi_b1, oi_b2, oi_b3, sem_b):
        cp.wait()


def kernel(tokens, word_embd_weight):
    tokens = tokens.reshape(B_, L_).astype(jnp.int32)
    tab = jnp.pad(word_embd_weight.T, ((0, DPAD_ - D_), (0, VP_ - VOCAB_))).T
    mesh = plsc.VectorSubcoreMesh(core_axis_name="c", subcore_axis_name="s")
    out = pl.kernel(
        _sc_body,
        out_type=jax.ShapeDtypeStruct((OSEG_, SEG_), jnp.float32),
        mesh=mesh,
        scratch_types=[
            pltpu.VMEM((BPW_, L_), jnp.int32),
            pltpu.VMEM((BPW_,), jnp.int32),
            pltpu.VMEM((BPW_, DPAD_), jnp.float32),
            pltpu.VMEM((D_, SEG_), jnp.float32),
            pltpu.VMEM((D_, SEG_), jnp.float32),
            pltpu.VMEM((SEG_,), jnp.int32),
            pltpu.VMEM((SEG_,), jnp.int32),
            pltpu.VMEM((D_ - 2 * SEG_,), jnp.int32),
            pltpu.VMEM((SEG_,), jnp.int32),
            pltpu.VMEM((SEG_,), jnp.int32),
            pltpu.VMEM((D_ - 2 * SEG_,), jnp.int32),
            pltpu.SemaphoreType.DMA,
            pltpu.SemaphoreType.DMA,
            pltpu.SemaphoreType.DMA,
        ],
        compiler_params=pltpu.CompilerParams(
            needs_layout_passes=False, use_tc_tiling_on_sc=False
        ),
        name="embed_gather_transpose",
    )(tab, tokens)
    out = out.reshape(D_, L_, NW_, BPW_)
    return out.transpose(2, 3, 0, 1).reshape(B_, D_, L_, 1)


# speedup vs baseline: 1.4603x; 1.0314x over previous
import jax
import jax.numpy as jnp
from jax import lax
from jax.experimental import pallas as pl
from jax.experimental.pallas import tpu as pltpu
from jax.experimental.pallas import tpu_sc as plsc

VOCAB_ = 100001
D_ = 300
B_ = 4096
L_ = 50
NC_ = 2
NS_ = 16
NW_ = NC_ * NS_
BPW_ = B_ // NW_
LANES_ = 16
SEG_ = 128
VP_ = (VOCAB_ + 7) // 8 * 8
DPAD_ = 304
OSEG_ = D_ * L_ * NW_
TAIL_D_ = 284
TAIL_SKIP_ = 4


def _sc_body(table_hbm, tokens_hbm, out_hbm, idx_v, idx_g, rows_p, rows_q,
             obuf_a, obuf_b, oi_a1, oi_a2, oi_a3, oi_b1, oi_b2, oi_b3,
             sem_g, sem_a, sem_b):
    c = lax.axis_index("c")
    s = lax.axis_index("s")
    wid = s * NC_ + c

    pltpu.sync_copy(tokens_hbm.at[pl.ds(wid * BPW_, BPW_)], idx_v)

    iota = lax.iota(jnp.int32, LANES_)
    iota_x_o = iota * (L_ * NW_)
    tail_mask = iota >= TAIL_SKIP_

    def scatter_cps(obuf, oi1, oi2, oi3, sem):
        return (
            pltpu.make_async_copy(obuf.at[pl.ds(0, SEG_)], out_hbm.at[oi1], sem),
            pltpu.make_async_copy(obuf.at[pl.ds(SEG_, SEG_)], out_hbm.at[oi2], sem),
            pltpu.make_async_copy(obuf.at[pl.ds(2 * SEG_, D_ - 2 * SEG_)], out_hbm.at[oi3], sem),
        )

    def step(l, drain, obuf, oi1, oi2, oi3, sem):
        for j in range(BPW_ // LANES_):
            t = plsc.load_gather(
                idx_v, [iota + j * LANES_, jnp.broadcast_to(l, (LANES_,))])
            idx_g[pl.ds(j * LANES_, LANES_)] = t
        half = BPW_ // 2
        cp_p = pltpu.async_copy(table_hbm.at[idx_g.at[pl.ds(0, half)]], rows_p, sem_g)
        cp_q = pltpu.async_copy(table_hbm.at[idx_g.at[pl.ds(half, half)]], rows_q, sem_g)

        @pl.when(drain)
        def _():
            for cp in scatter_cps(obuf, oi1, oi2, oi3, sem):
                cp.wait()

        obase = l * NW_ + wid
        for cc in range(8):
            oi1[pl.ds(cc * LANES_, LANES_)] = iota_x_o + (cc * LANES_ * L_ * NW_ + obase)
        for cc in range(8, 16):
            oi2[pl.ds((cc - 8) * LANES_, LANES_)] = iota_x_o + (cc * LANES_ * L_ * NW_ + obase)
        for cc, off in ((16, 0), (17, 16)):
            oi3[pl.ds(off, LANES_)] = iota_x_o + (cc * LANES_ * L_ * NW_ + obase)
        oi3[pl.ds(TAIL_D_ - 2 * SEG_, LANES_)] = iota_x_o + (TAIL_D_ * L_ * NW_ + obase)

        def make_t_body(rows_v, col0):
            def t_body(jb, _):
                col = jnp.broadcast_to(jb + col0, (LANES_,))
                for cc in range(18):
                    v = rows_v[jb, pl.ds(cc * LANES_, LANES_)]
                    plsc.store_scatter(obuf, [iota + cc * LANES_, col], v)
                v = rows_v[jb, pl.ds(TAIL_D_, LANES_)]
                plsc.store_scatter(obuf, [iota + TAIL_D_, col], v, mask=tail_mask)
                return 0
            return t_body

        cp_p.wait()
        lax.fori_loop(0, half, make_t_body(rows_p, 0), 0)
        cp_q.wait()
        lax.fori_loop(0, half, make_t_body(rows_q, half), 0)

        for cp in scatter_cps(obuf, oi1, oi2, oi3, sem):
            cp.start()

    def pair_body(i, _):
        step(2 * i, i > 0, obuf_a, oi_a1, oi_a2, oi_a3, sem_a)
        step(2 * i + 1, i > 0, obuf_b, oi_b1, oi_b2, oi_b3, sem_b)
        return 0

    lax.fori_loop(0, L_ // 2, pair_body, 0)
    for cp in scatter_cps(obuf_a, oi_a1, oi_a2, oi_a3, sem_a):
        cp.wait()
    for cp in scatter_cps(obuf_b, oi_b1, oi_b2, oi_b3, sem_b):
        cp.wait()


def kernel(tokens, word_embd_weight):
    tokens = tokens.reshape(B_, L_).astype(jnp.int32)
    tab = jnp.pad(word_embd_weight.T, ((0, DPAD_ - D_), (0, VP_ - VOCAB_))).T
    mesh = plsc.VectorSubcoreMesh(core_axis_name="c", subcore_axis_name="s")
    out = pl.kernel(
        _sc_body,
        out_type=jax.ShapeDtypeStruct((OSEG_, SEG_), jnp.float32),
        mesh=mesh,
        scratch_types=[
            pltpu.VMEM((BPW_, L_), jnp.int32),
            pltpu.VMEM((BPW_,), jnp.int32),
            pltpu.VMEM((BPW_ // 2, DPAD_), jnp.float32),
            pltpu.VMEM((BPW_ // 2, DPAD_), jnp.float32),
            pltpu.VMEM((D_, SEG_), jnp.float32),
            pltpu.VMEM((D_, SEG_), jnp.float32),
            pltpu.VMEM((SEG_,), jnp.int32),
            pltpu.VMEM((SEG_,), jnp.int32),
            pltpu.VMEM((D_ - 2 * SEG_,), jnp.int32),
            pltpu.VMEM((SEG_,), jnp.int32),
            pltpu.VMEM((SEG_,), jnp.int32),
            pltpu.VMEM((D_ - 2 * SEG_,), jnp.int32),
            pltpu.SemaphoreType.DMA,
            pltpu.SemaphoreType.DMA,
            pltpu.SemaphoreType.DMA,
        ],
        compiler_params=pltpu.CompilerParams(
            needs_layout_passes=False, use_tc_tiling_on_sc=False
        ),
        name="embed_gather_transpose",
    )(tab, tokens)
    out = out.reshape(D_, L_, NW_, BPW_)
    return out.transpose(2, 3, 0, 1).reshape(B_, D_, L_, 1)
